# baseline (speedup 1.0000x reference)
import functools

import jax
import jax.numpy as jnp
from jax import lax
from jax.experimental import pallas as pl
from jax.experimental.pallas import tpu as pltpu
from jax.experimental.pallas import tpu_sc as plsc

B = 16
N = 2048
K = 16
R1 = 512
R5 = 512


def _dotT(x, w):
    return lax.dot_general(x, w, (((1,), (1,)), ((), ())),
                           preferred_element_type=jnp.float32)


def _k1_body(ptsr_ref, ptst_ref, w1_ref, b1_ref, w2_ref, b2_ref, w3_ref,
             b3_ref, idx_ref, f1_ref):
    b = pl.program_id(0)
    A = ptsr_ref[0]
    Bt = ptst_ref[0]
    t = lax.dot_general(A, Bt, (((1,), (0,)), ((), ())),
                        preferred_element_type=jnp.float32)
    inner = -2.0 * t
    xn = jnp.sum(A * A, axis=1, keepdims=True)
    xm = jnp.sum(Bt * Bt, axis=0, keepdims=True)
    D = (-xn - inner) - xm

    iota = lax.broadcasted_iota(jnp.int32, (R1, N), 1)
    neg_inf = jnp.float32(-jnp.inf)
    cols = []
    sel = []
    for j in range(K):
        i = jnp.argmax(D, axis=1, keepdims=True)
        cols.append(i.astype(jnp.int32))
        hit = iota == i
        if j < 2:
            oh = hit.astype(jnp.float32)
            sel.append(lax.dot_general(oh, Bt, (((1,), (1,)), ((), ())),
                                       preferred_element_type=jnp.float32))
        if j < K - 1:
            D = jnp.where(hit, neg_inf, D)

    idx_ref[...] = jnp.concatenate(cols, axis=1) + b * N

    x0, x1 = sel
    cov = jnp.concatenate([x0[:, 0:1] * x1, x0[:, 1:2] * x1, x0[:, 2:3] * x1],
                          axis=1)
    f = jnp.concatenate([A, cov], axis=1)
    h = jnp.maximum(_dotT(f, w1_ref[...]) + b1_ref[...][None, :], 0.0)
    h = jnp.maximum(_dotT(h, w2_ref[...]) + b2_ref[...][None, :], 0.0)
    h = jnp.maximum(_dotT(h, w3_ref[...]) + b3_ref[...][None, :], 0.0)
    f1_ref[...] = h


def _knn_mlp1(nb, pts, ptst, W1, b1, W2, b2, W3, b3):
    nrb = N // R1
    grid = (nb, nrb)
    wspec = lambda shp: pl.BlockSpec(shp, lambda b_, r_: (0,) * len(shp))
    return pl.pallas_call(
        _k1_body,
        grid=grid,
        in_specs=[
            pl.BlockSpec((1, R1, 3), lambda b_, r_: (b_, r_, 0)),
            pl.BlockSpec((1, 3, N), lambda b_, r_: (b_, 0, 0)),
            wspec((64, 12)), wspec((64,)),
            wspec((64, 64)), wspec((64,)),
            wspec((64, 64)), wspec((64,)),
        ],
        out_specs=[
            pl.BlockSpec((R1, K), lambda b_, r_: (b_ * nrb + r_, 0)),
            pl.BlockSpec((R1, 64), lambda b_, r_: (b_ * nrb + r_, 0)),
        ],
        out_shape=[
            jax.ShapeDtypeStruct((nb * N, K), jnp.int32),
            jax.ShapeDtypeStruct((nb * N, 64), jnp.float32),
        ],
    )(pts, ptst, W1, b1, W2, b2, W3, b3)


def _make_pool(Dch, nb):
    info = plsc.get_sparse_core_info()
    NC, NS = info.num_cores, info.num_subcores
    NW = NC * NS
    ROWS_W = (nb * N) // NW
    SEG = 256
    BLK = 8
    NSEG = ROWS_W // SEG
    NBLK = SEG // BLK
    NBUF = 4
    mesh = plsc.VectorSubcoreMesh(core_axis_name="c", subcore_axis_name="s")

    @functools.partial(
        pl.kernel, mesh=mesh,
        compiler_params=pltpu.CompilerParams(use_tc_tiling_on_sc=False),
        out_type=jax.ShapeDtypeStruct((nb * N, Dch), jnp.float32),
        scratch_types=[
            pltpu.VMEM((NBLK, BLK * K), jnp.int32),
            pltpu.VMEM((NBUF, BLK * K, Dch), jnp.float32),
            pltpu.VMEM((SEG, Dch), jnp.float32),
            pltpu.SemaphoreType.DMA,
            pltpu.SemaphoreType.DMA,
            pltpu.SemaphoreType.DMA,
            pltpu.SemaphoreType.DMA,
        ],
    )
    def pool(table_hbm, idx_hbm, out_hbm, idx_v, buf_v, out_v, sm0, sm1, sm2,
             sm3):
        sems = (sm0, sm1, sm2, sm3)
        wid = lax.axis_index("s") * NC + lax.axis_index("c")

        def seg_body(s, _):
            base_out = pl.multiple_of(wid * ROWS_W + s * SEG, SEG)
            idx_row0 = pl.multiple_of(base_out // BLK, SEG // BLK)
            pltpu.sync_copy(idx_hbm.at[pl.ds(idx_row0, NBLK)], idx_v)
            for q in range(NBUF):
                pltpu.async_copy(table_hbm.at[idx_v.at[q]], buf_v.at[q],
                                 sems[q])

            def grp_body(g, _):
                for q in range(NBUF):
                    jb = g * NBUF + q
                    pltpu.make_async_copy(table_hbm.at[idx_v.at[jb]],
                                          buf_v.at[q], sems[q]).wait()
                    for r in range(BLK):
                        orow = jb * BLK + r
                        for c in range(Dch // 16):
                            cs = pl.ds(c * 16, 16)
                            acc = buf_v[q, r * K, cs]
                            for jn in range(1, K):
                                acc = jnp.maximum(acc, buf_v[q, r * K + jn, cs])
                            out_v[orow, cs] = acc

                    @pl.when(jb + NBUF < NBLK)
                    def _():
                        pltpu.async_copy(table_hbm.at[idx_v.at[jb + NBUF]],
                                         buf_v.at[q], sems[q])
                return 0

            lax.fori_loop(0, NBLK // NBUF, grp_body, 0)
            pltpu.sync_copy(out_v, out_hbm.at[pl.ds(base_out, SEG)])
            return 0

        lax.fori_loop(0, NSEG, seg_body, 0)

    return pool


def _k3_body(x_ref, l1_ref, bl1_ref, c1_ref, bc1_ref, o_ref):
    y = _dotT(x_ref[...], l1_ref[...]) + bl1_ref[...][None, :]
    o_ref[...] = jnp.maximum(_dotT(y, c1_ref[...]) + bc1_ref[...][None, :],
                             0.0)


def _mid_mlp(nb, x, L1, bl1, C1, bc1):
    R = 1024
    wspec = lambda shp: pl.BlockSpec(shp, lambda i: (0,) * len(shp))
    return pl.pallas_call(
        _k3_body,
        grid=((nb * N) // R,),
        in_specs=[
            pl.BlockSpec((R, 64), lambda i: (i, 0)),
            wspec((64, 64)), wspec((64,)),
            wspec((128, 64)), wspec((128,)),
        ],
        out_specs=pl.BlockSpec((R, 128), lambda i: (i, 0)),
        out_shape=jax.ShapeDtypeStruct((nb * N, 128), jnp.float32),
    )(x, L1, bl1, C1, bc1)


def _k5_body(x_ref, l2_ref, bl2_ref, c2_ref, bc2_ref, o_ref):
    nb_ = pl.program_id(1)
    y = _dotT(x_ref[...], l2_ref[...]) + bl2_ref[...][None, :]
    z = _dotT(y, c2_ref[...]) + bc2_ref[...][None, :]
    mx = jnp.max(z, axis=0, keepdims=True)

    @pl.when(nb_ == 0)
    def _():
        o_ref[0] = mx

    @pl.when(nb_ != 0)
    def _():
        o_ref[0] = jnp.maximum(o_ref[0], mx)


def _tail_max(nb, x, L2, bl2, C2, bc2):
    nrb = N // R5
    wspec = lambda shp: pl.BlockSpec(shp, lambda b_, i_: (0,) * len(shp))
    return pl.pallas_call(
        _k5_body,
        grid=(nb, nrb),
        in_specs=[
            pl.BlockSpec((R5, 128), lambda b_, i_: (b_ * nrb + i_, 0)),
            wspec((128, 128)), wspec((128,)),
            wspec((1024, 128)), wspec((1024,)),
        ],
        out_specs=pl.BlockSpec((1, 1, 1024), lambda b_, i_: (b_, 0, 0)),
        out_shape=jax.ShapeDtypeStruct((nb, 1, 1024), jnp.float32),
    )(x, L2, bl2, C2, bc2)


def _k6_body(g_ref, m1_ref, bm1_ref, m2_ref, bm2_ref, o_ref):
    h = jnp.maximum(_dotT(g_ref[...], m1_ref[...]) + bm1_ref[...][None, :],
                    0.0)
    o_ref[...] = _dotT(h, m2_ref[...]) + bm2_ref[...][None, :]


def _head(g, M1, bm1, M2, bm2):
    wspec = lambda shp: pl.BlockSpec(shp, lambda: (0,) * len(shp))
    return pl.pallas_call(
        _k6_body,
        in_specs=[
            wspec((B, 1024)),
            wspec((512, 1024)), wspec((512,)),
            wspec((512, 512)), wspec((512,)),
        ],
        out_specs=wspec((B, 512)),
        out_shape=jax.ShapeDtypeStruct((B, 512), jnp.float32),
    )(g, M1, bm1, M2, bm2)


def kernel(pts, W1, b1, W2, b2, W3, b3, L1, bl1, C1, bc1, L2, bl2, C2, bc2,
           M1, bm1, M2, bm2):
    ptst = jnp.transpose(pts, (0, 2, 1))
    idx, f1 = _knn_mlp1(B, pts, ptst, W1, b1, W2, b2, W3, b3)
    idx2d = idx.reshape((B * N * K) // 128, 128)
    p1 = _make_pool(64, B)(f1, idx2d)
    f2 = _mid_mlp(B, p1, L1, bl1, C1, bc1)
    p2 = _make_pool(128, B)(f2, idx2d)
    g = _tail_max(B, p2, L2, bl2, C2, bc2).reshape(B, 1024)
    return _head(g, M1, bm1, M2, bm2)


# speedup vs baseline: 1.1409x; 1.1409x over previous
import functools

import jax
import jax.numpy as jnp
from jax import lax
from jax.experimental import pallas as pl
from jax.experimental.pallas import tpu as pltpu
from jax.experimental.pallas import tpu_sc as plsc

B = 16
N = 2048
K = 16
R1 = 512
R5 = 512


def _dotT(x, w):
    return lax.dot_general(x, w, (((1,), (1,)), ((), ())),
                           preferred_element_type=jnp.float32)


def _k1_body(ptsr_ref, ptst_ref, w1_ref, b1_ref, w2_ref, b2_ref, w3_ref,
             b3_ref, idx_ref, f1_ref):
    b = pl.program_id(0)
    A = ptsr_ref[0]
    Bt = ptst_ref[0]
    t = lax.dot_general(A, Bt, (((1,), (0,)), ((), ())),
                        preferred_element_type=jnp.float32)
    inner = -2.0 * t
    xn = jnp.sum(A * A, axis=1, keepdims=True)
    xm = jnp.sum(Bt * Bt, axis=0, keepdims=True)
    D = (-xn - inner) - xm

    iota = lax.broadcasted_iota(jnp.int32, (R1, N), 1)
    neg_inf = jnp.float32(-jnp.inf)
    cols = []
    sel = []
    for j in range(K):
        i = jnp.argmax(D, axis=1, keepdims=True)
        cols.append(i.astype(jnp.int32))
        hit = iota == i
        if j < 2:
            oh = hit.astype(jnp.float32)
            sel.append(lax.dot_general(oh, Bt, (((1,), (1,)), ((), ())),
                                       preferred_element_type=jnp.float32))
        if j < K - 1:
            D = jnp.where(hit, neg_inf, D)

    idx_ref[...] = jnp.concatenate(cols, axis=1) + b * N

    x0, x1 = sel
    cov = jnp.concatenate([x0[:, 0:1] * x1, x0[:, 1:2] * x1, x0[:, 2:3] * x1],
                          axis=1)
    f = jnp.concatenate([A, cov], axis=1)
    h = jnp.maximum(_dotT(f, w1_ref[...]) + b1_ref[...][None, :], 0.0)
    h = jnp.maximum(_dotT(h, w2_ref[...]) + b2_ref[...][None, :], 0.0)
    h = jnp.maximum(_dotT(h, w3_ref[...]) + b3_ref[...][None, :], 0.0)
    f1_ref[...] = h


def _knn_mlp1(nb, pts, ptst, W1, b1, W2, b2, W3, b3):
    nrb = N // R1
    grid = (nb, nrb)
    wspec = lambda shp: pl.BlockSpec(shp, lambda b_, r_: (0,) * len(shp))
    return pl.pallas_call(
        _k1_body,
        grid=grid,
        in_specs=[
            pl.BlockSpec((1, R1, 3), lambda b_, r_: (b_, r_, 0)),
            pl.BlockSpec((1, 3, N), lambda b_, r_: (b_, 0, 0)),
            wspec((64, 12)), wspec((64,)),
            wspec((64, 64)), wspec((64,)),
            wspec((64, 64)), wspec((64,)),
        ],
        out_specs=[
            pl.BlockSpec((R1, K), lambda b_, r_: (b_ * nrb + r_, 0)),
            pl.BlockSpec((R1, 64), lambda b_, r_: (b_ * nrb + r_, 0)),
        ],
        out_shape=[
            jax.ShapeDtypeStruct((nb * N, K), jnp.int32),
            jax.ShapeDtypeStruct((nb * N, 64), jnp.float32),
        ],
    )(pts, ptst, W1, b1, W2, b2, W3, b3)


def _make_pool(Dch, nb):
    info = plsc.get_sparse_core_info()
    NC, NS = info.num_cores, info.num_subcores
    NW = NC * NS
    ROWS_W = (nb * N) // NW
    SEG = 256
    BLK = 8
    NSEG = ROWS_W // SEG
    NBLK = SEG // BLK
    NBUF = 2
    mesh = plsc.VectorSubcoreMesh(core_axis_name="c", subcore_axis_name="s")

    @functools.partial(
        pl.kernel, mesh=mesh,
        compiler_params=pltpu.CompilerParams(use_tc_tiling_on_sc=False),
        out_type=jax.ShapeDtypeStruct((nb * N, Dch), jnp.float32),
        scratch_types=[
            pltpu.VMEM((NBLK, BLK * K), jnp.int32),
            pltpu.VMEM((NBUF, BLK * K, Dch), jnp.float32),
            pltpu.VMEM((SEG, Dch), jnp.float32),
            pltpu.SemaphoreType.DMA,
            pltpu.SemaphoreType.DMA,
            pltpu.SemaphoreType.DMA,
            pltpu.SemaphoreType.DMA,
        ],
    )
    def pool(table_hbm, idx_hbm, out_hbm, idx_v, buf_v, out_v, sm0, sm1, sm2,
             sm3):
        sems = (sm0, sm1, sm2, sm3)
        wid = lax.axis_index("s") * NC + lax.axis_index("c")

        def seg_body(s, _):
            base_out = pl.multiple_of(wid * ROWS_W + s * SEG, SEG)
            idx_row0 = pl.multiple_of(base_out // BLK, SEG // BLK)
            pltpu.sync_copy(idx_hbm.at[pl.ds(idx_row0, NBLK)], idx_v)
            for q in range(NBUF):
                pltpu.async_copy(table_hbm.at[idx_v.at[q]], buf_v.at[q],
                                 sems[q])

            def grp_body(g, _):
                for q in range(NBUF):
                    jb = g * NBUF + q
                    pltpu.make_async_copy(table_hbm.at[idx_v.at[jb]],
                                          buf_v.at[q], sems[q]).wait()
                    for r in range(BLK):
                        orow = jb * BLK + r
                        for c in range(Dch // 16):
                            cs = pl.ds(c * 16, 16)
                            acc = buf_v[q, r * K, cs]
                            for jn in range(1, K):
                                acc = jnp.maximum(acc, buf_v[q, r * K + jn, cs])
                            out_v[orow, cs] = acc

                    @pl.when(jb + NBUF < NBLK)
                    def _():
                        pltpu.async_copy(table_hbm.at[idx_v.at[jb + NBUF]],
                                         buf_v.at[q], sems[q])
                return 0

            lax.fori_loop(0, NBLK // NBUF, grp_body, 0)
            pltpu.sync_copy(out_v, out_hbm.at[pl.ds(base_out, SEG)])
            return 0

        lax.fori_loop(0, NSEG, seg_body, 0)

    return pool


def _k3_body(x_ref, l1_ref, bl1_ref, c1_ref, bc1_ref, o_ref):
    y = _dotT(x_ref[...], l1_ref[...]) + bl1_ref[...][None, :]
    o_ref[...] = jnp.maximum(_dotT(y, c1_ref[...]) + bc1_ref[...][None, :],
                             0.0)


def _mid_mlp(nb, x, L1, bl1, C1, bc1):
    R = 1024
    wspec = lambda shp: pl.BlockSpec(shp, lambda i: (0,) * len(shp))
    return pl.pallas_call(
        _k3_body,
        grid=((nb * N) // R,),
        in_specs=[
            pl.BlockSpec((R, 64), lambda i: (i, 0)),
            wspec((64, 64)), wspec((64,)),
            wspec((128, 64)), wspec((128,)),
        ],
        out_specs=pl.BlockSpec((R, 128), lambda i: (i, 0)),
        out_shape=jax.ShapeDtypeStruct((nb * N, 128), jnp.float32),
    )(x, L1, bl1, C1, bc1)


def _k5_body(x_ref, l2_ref, bl2_ref, c2_ref, bc2_ref, o_ref):
    nb_ = pl.program_id(1)
    y = _dotT(x_ref[...], l2_ref[...]) + bl2_ref[...][None, :]
    z = _dotT(y, c2_ref[...]) + bc2_ref[...][None, :]
    mx = jnp.max(z, axis=0, keepdims=True)

    @pl.when(nb_ == 0)
    def _():
        o_ref[0] = mx

    @pl.when(nb_ != 0)
    def _():
        o_ref[0] = jnp.maximum(o_ref[0], mx)


def _tail_max(nb, x, L2, bl2, C2, bc2):
    nrb = N // R5
    wspec = lambda shp: pl.BlockSpec(shp, lambda b_, i_: (0,) * len(shp))
    return pl.pallas_call(
        _k5_body,
        grid=(nb, nrb),
        in_specs=[
            pl.BlockSpec((R5, 128), lambda b_, i_: (b_ * nrb + i_, 0)),
            wspec((128, 128)), wspec((128,)),
            wspec((1024, 128)), wspec((1024,)),
        ],
        out_specs=pl.BlockSpec((1, 1, 1024), lambda b_, i_: (b_, 0, 0)),
        out_shape=jax.ShapeDtypeStruct((nb, 1, 1024), jnp.float32),
    )(x, L2, bl2, C2, bc2)


def _k6_body(g_ref, m1_ref, bm1_ref, m2_ref, bm2_ref, o_ref):
    h = jnp.maximum(_dotT(g_ref[...], m1_ref[...]) + bm1_ref[...][None, :],
                    0.0)
    o_ref[...] = _dotT(h, m2_ref[...]) + bm2_ref[...][None, :]


def _head(g, M1, bm1, M2, bm2):
    wspec = lambda shp: pl.BlockSpec(shp, lambda: (0,) * len(shp))
    return pl.pallas_call(
        _k6_body,
        in_specs=[
            wspec((B, 1024)),
            wspec((512, 1024)), wspec((512,)),
            wspec((512, 512)), wspec((512,)),
        ],
        out_specs=wspec((B, 512)),
        out_shape=jax.ShapeDtypeStruct((B, 512), jnp.float32),
    )(g, M1, bm1, M2, bm2)


def kernel(pts, W1, b1, W2, b2, W3, b3, L1, bl1, C1, bc1, L2, bl2, C2, bc2,
           M1, bm1, M2, bm2):
    NHALF = 2
    nb = B // NHALF
    pool64 = _make_pool(64, nb)
    pool128 = _make_pool(128, nb)
    gs = []
    for h in range(NHALF):
        p_h = lax.slice_in_dim(pts, h * nb, (h + 1) * nb, axis=0)
        ptst_h = jnp.transpose(p_h, (0, 2, 1))
        idx, f1 = _knn_mlp1(nb, p_h, ptst_h, W1, b1, W2, b2, W3, b3)
        idx2d = idx.reshape((nb * N * K) // 128, 128)
        p1 = pool64(f1, idx2d)
        f2 = _mid_mlp(nb, p1, L1, bl1, C1, bc1)
        p2 = pool128(f2, idx2d)
        gs.append(_tail_max(nb, p2, L2, bl2, C2, bc2).reshape(nb, 1024))
    g = jnp.concatenate(gs, axis=0)
    return _head(g, M1, bm1, M2, bm2)
